# Initial kernel scaffold; baseline (speedup 1.0000x reference)
import jax
import jax.numpy as jnp
from jax.experimental import pallas as pl


def kernel(x, edge_index, W1, b1, W2, b2, Wl, bl):
    raise NotImplementedError("write your pallas kernel here")


# speedup vs baseline: 14.2736x; 14.2736x over previous
import functools

import jax
import jax.numpy as jnp
from jax import lax
from jax.experimental import pallas as pl
from jax.experimental.pallas import tpu as pltpu
from jax.experimental.pallas import tpu_sc as plsc

_N = 10000
_E = 320000
_NC = 2
_NS = 16
_NW = _NC * _NS
_BATCH = 128
_NB = -(-_E // (_NW * _BATCH))
_EPT = _NB * _BATCH
_E_PAD = _EPT * _NW
_ROWS_PT = 632
_N_ACC = _NS * _ROWS_PT

_mesh = plsc.VectorSubcoreMesh(
    core_axis_name="c", subcore_axis_name="s", num_cores=_NC, num_subcores=_NS
)


def _make_edge_scatter(d):
  @functools.partial(
      pl.kernel,
      out_type=jax.ShapeDtypeStruct((_NC, _N_ACC, d), jnp.float32),
      mesh=_mesh,
      scratch_types=[
          pltpu.VMEM((_NB, _BATCH), jnp.int32),
          pltpu.VMEM((_NB, _BATCH), jnp.int32),
          pltpu.VMEM((_BATCH, d), jnp.float32),
          pltpu.VMEM_SHARED((_N_ACC, d), jnp.float32),
          pltpu.SemaphoreType.DMA,
      ],
      compiler_params=pltpu.CompilerParams(use_tc_tiling_on_sc=False),
  )
  def k(g_hbm, src_hbm, dst_hbm, zero_hbm, out_hbm, src_v, dst_v, rows_v, acc, sem):
    cid = lax.axis_index("c")
    sid = lax.axis_index("s")
    wid = cid * _NS + sid
    pltpu.sync_copy(src_hbm.at[wid], src_v)
    pltpu.sync_copy(dst_hbm.at[wid], dst_v)
    pltpu.sync_copy(zero_hbm, acc.at[pl.ds(sid * _ROWS_PT, _ROWS_PT)])
    plsc.subcore_barrier()

    def step(j, carry):
      pltpu.async_copy(g_hbm.at[src_v.at[j]], rows_v, sem).wait()
      pltpu.sync_copy(rows_v, acc.at[dst_v.at[j]], add=True)
      return carry

    lax.fori_loop(0, _NB, step, 0)
    plsc.subcore_barrier()
    pltpu.sync_copy(
        acc.at[pl.ds(sid * _ROWS_PT, _ROWS_PT)],
        out_hbm.at[cid, pl.ds(sid * _ROWS_PT, _ROWS_PT)],
    )

  return k


_edge_scatter_128 = _make_edge_scatter(128)
_edge_scatter_64 = _make_edge_scatter(64)


@functools.partial(
    pl.kernel,
    out_type=jax.ShapeDtypeStruct((_NC, _N_ACC, 16), jnp.float32),
    mesh=_mesh,
    scratch_types=[
        pltpu.VMEM((_NB, _BATCH), jnp.int32),
        pltpu.VMEM((_BATCH, 16), jnp.float32),
        pltpu.VMEM_SHARED((_N_ACC, 16), jnp.float32),
    ],
    compiler_params=pltpu.CompilerParams(use_tc_tiling_on_sc=False),
)
def _deg_count(dst_hbm, ones_hbm, zero_hbm, out_hbm, dst_v, ones_v, acc):
  cid = lax.axis_index("c")
  sid = lax.axis_index("s")
  wid = cid * _NS + sid
  pltpu.sync_copy(dst_hbm.at[wid], dst_v)
  pltpu.sync_copy(ones_hbm, ones_v)
  pltpu.sync_copy(zero_hbm, acc.at[pl.ds(sid * _ROWS_PT, _ROWS_PT)])
  plsc.subcore_barrier()

  def step(j, carry):
    pltpu.sync_copy(ones_v, acc.at[dst_v.at[j]], add=True)
    return carry

  lax.fori_loop(0, _NB, step, 0)
  plsc.subcore_barrier()
  pltpu.sync_copy(
      acc.at[pl.ds(sid * _ROWS_PT, _ROWS_PT)],
      out_hbm.at[cid, pl.ds(sid * _ROWS_PT, _ROWS_PT)],
  )


_BN = 2000


def _stage_a_body(degp_ref, x_ref, w1_ref, g1_ref, dinv_ref):
  deg = degp_ref[0, :, 0:1] + degp_ref[1, :, 0:1] + 1.0
  dinv = lax.rsqrt(deg)
  dinv_ref[...] = jnp.broadcast_to(dinv, (_BN, 16))
  p1 = jnp.dot(x_ref[...], w1_ref[...], preferred_element_type=jnp.float32)
  g1_ref[...] = dinv * p1


def _stage_a(degp, x, w1):
  return pl.pallas_call(
      _stage_a_body,
      grid=(_N // _BN,),
      in_specs=[
          pl.BlockSpec((_NC, _BN, 16), lambda i: (0, i, 0)),
          pl.BlockSpec((_BN, 128), lambda i: (i, 0)),
          pl.BlockSpec((128, 128), lambda i: (0, 0)),
      ],
      out_specs=[
          pl.BlockSpec((_BN, 128), lambda i: (i, 0)),
          pl.BlockSpec((_BN, 16), lambda i: (i, 0)),
      ],
      out_shape=[
          jax.ShapeDtypeStruct((_N, 128), jnp.float32),
          jax.ShapeDtypeStruct((_N, 16), jnp.float32),
      ],
  )(degp, x, w1)


def _stage_b_body(s1_ref, g1_ref, dinv_ref, b1_ref, w2_ref, h_ref, g2_ref):
  dinv = dinv_ref[:, 0:1]
  h = jnp.maximum(dinv * (s1_ref[0] + s1_ref[1] + g1_ref[...]) + b1_ref[...], 0.0)
  h_ref[...] = h
  g2_ref[...] = dinv * jnp.dot(h, w2_ref[...], preferred_element_type=jnp.float32)


def _stage_b(s1, g1, dinv, b1, w2):
  return pl.pallas_call(
      _stage_b_body,
      grid=(_N // _BN,),
      in_specs=[
          pl.BlockSpec((_NC, _BN, 128), lambda i: (0, i, 0)),
          pl.BlockSpec((_BN, 128), lambda i: (i, 0)),
          pl.BlockSpec((_BN, 16), lambda i: (i, 0)),
          pl.BlockSpec((1, 128), lambda i: (0, 0)),
          pl.BlockSpec((128, 64), lambda i: (0, 0)),
      ],
      out_specs=[
          pl.BlockSpec((_BN, 128), lambda i: (i, 0)),
          pl.BlockSpec((_BN, 64), lambda i: (i, 0)),
      ],
      out_shape=[
          jax.ShapeDtypeStruct((_N, 128), jnp.float32),
          jax.ShapeDtypeStruct((_N, 64), jnp.float32),
      ],
  )(s1, g1, dinv, b1, w2)


def _stage_c_body(s2_ref, g2_ref, dinv_ref, b2_ref, x_ref, h_ref, wl_ref, bl_ref, out_ref):
  dinv = dinv_ref[:, 0:1]
  h2 = dinv * (s2_ref[0] + s2_ref[1] + g2_ref[...]) + b2_ref[...]
  wl = wl_ref[...]
  logits = (
      jnp.dot(x_ref[...], wl[0:128], preferred_element_type=jnp.float32)
      + jnp.dot(h_ref[...], wl[128:256], preferred_element_type=jnp.float32)
      + jnp.dot(h2, wl[256:320], preferred_element_type=jnp.float32)
      + bl_ref[...]
  )
  m = jnp.max(logits, axis=-1, keepdims=True)
  z = logits - m
  lse = jnp.log(jnp.sum(jnp.exp(z), axis=-1, keepdims=True))
  out_ref[...] = z - lse


def _stage_c(s2, g2, dinv, b2, x, h, wl, bl):
  return pl.pallas_call(
      _stage_c_body,
      grid=(_N // _BN,),
      in_specs=[
          pl.BlockSpec((_NC, _BN, 64), lambda i: (0, i, 0)),
          pl.BlockSpec((_BN, 64), lambda i: (i, 0)),
          pl.BlockSpec((_BN, 16), lambda i: (i, 0)),
          pl.BlockSpec((1, 64), lambda i: (0, 0)),
          pl.BlockSpec((_BN, 128), lambda i: (i, 0)),
          pl.BlockSpec((_BN, 128), lambda i: (i, 0)),
          pl.BlockSpec((320, 64), lambda i: (0, 0)),
          pl.BlockSpec((1, 64), lambda i: (0, 0)),
      ],
      out_specs=pl.BlockSpec((_BN, 64), lambda i: (i, 0)),
      out_shape=jax.ShapeDtypeStruct((_N, 64), jnp.float32),
  )(s2, g2, dinv, b2, x, h, wl, bl)


def kernel(x, edge_index, W1, b1, W2, b2, Wl, bl):
  src = edge_index[0]
  dst = edge_index[1]
  pad = _E_PAD - _E
  srcb = jnp.concatenate([src, jnp.zeros((pad,), jnp.int32)]).reshape(_NW, _NB, _BATCH)
  dstb = jnp.concatenate([dst, jnp.full((pad,), _N, jnp.int32)]).reshape(_NW, _NB, _BATCH)
  zeros128 = jnp.zeros((_ROWS_PT, 128), jnp.float32)
  zeros64 = jnp.zeros((_ROWS_PT, 64), jnp.float32)
  zeros16 = jnp.zeros((_ROWS_PT, 16), jnp.float32)
  ones16 = jnp.ones((_BATCH, 16), jnp.float32)

  degp = _deg_count(dstb, ones16, zeros16)[:, :_N]
  g1, dinv = _stage_a(degp, x, W1)
  s1 = _edge_scatter_128(g1, srcb, dstb, zeros128)[:, :_N]
  h, g2 = _stage_b(s1, g1, dinv, b1.reshape(1, -1), W2)
  s2 = _edge_scatter_64(g2, srcb, dstb, zeros64)[:, :_N]
  return _stage_c(s2, g2, dinv, b2.reshape(1, -1), x, h, Wl, bl.reshape(1, -1))


# speedup vs baseline: 14.4242x; 1.0106x over previous
import functools

import jax
import jax.numpy as jnp
from jax import lax
from jax.experimental import pallas as pl
from jax.experimental.pallas import tpu as pltpu
from jax.experimental.pallas import tpu_sc as plsc

_N = 10000
_E = 320000
_NC = 2
_NS = 16
_NW = _NC * _NS
_ROWS_PT = 632
_N_ACC = _NS * _ROWS_PT

_B128, _NB128 = 112, 90
_B64, _NB64 = 128, 80
_mesh = plsc.VectorSubcoreMesh(
    core_axis_name="c", subcore_axis_name="s", num_cores=_NC, num_subcores=_NS
)


def _make_edge_scatter(d, batch, nb):
  @functools.partial(
      pl.kernel,
      out_type=jax.ShapeDtypeStruct((_NC, _N_ACC, d), jnp.float32),
      mesh=_mesh,
      scratch_types=[
          pltpu.VMEM((nb, batch), jnp.int32),
          pltpu.VMEM((nb, batch), jnp.int32),
          pltpu.VMEM((2, batch, d), jnp.float32),
          pltpu.VMEM_SHARED((_N_ACC, d), jnp.float32),
          pltpu.SemaphoreType.DMA,
          pltpu.SemaphoreType.DMA,
      ],
      compiler_params=pltpu.CompilerParams(use_tc_tiling_on_sc=False),
  )
  def k(g_hbm, src_hbm, dst_hbm, zero_hbm, out_hbm, src_v, dst_v, rows_v, acc,
        sem0, sem1):
    cid = lax.axis_index("c")
    sid = lax.axis_index("s")
    wid = cid * _NS + sid
    pltpu.sync_copy(src_hbm.at[wid], src_v)
    pltpu.sync_copy(dst_hbm.at[wid], dst_v)
    pltpu.sync_copy(zero_hbm, acc.at[pl.ds(sid * _ROWS_PT, _ROWS_PT)])
    plsc.subcore_barrier()

    pltpu.async_copy(g_hbm.at[src_v.at[0]], rows_v.at[0], sem0)

    def step(i, carry):
      j0 = 2 * i
      j1 = j0 + 1
      pltpu.make_async_copy(g_hbm.at[src_v.at[j0]], rows_v.at[0], sem0).wait()
      pltpu.async_copy(g_hbm.at[src_v.at[j1]], rows_v.at[1], sem1)
      pltpu.sync_copy(rows_v.at[0], acc.at[dst_v.at[j0]], add=True)
      pltpu.make_async_copy(g_hbm.at[src_v.at[j1]], rows_v.at[1], sem1).wait()

      @pl.when(i + 1 < nb // 2)
      def _():
        pltpu.async_copy(g_hbm.at[src_v.at[j0 + 2]], rows_v.at[0], sem0)

      pltpu.sync_copy(rows_v.at[1], acc.at[dst_v.at[j1]], add=True)
      return carry

    lax.fori_loop(0, nb // 2, step, 0)
    plsc.subcore_barrier()
    pltpu.sync_copy(
        acc.at[pl.ds(sid * _ROWS_PT, _ROWS_PT)],
        out_hbm.at[cid, pl.ds(sid * _ROWS_PT, _ROWS_PT)],
    )

  return k


_edge_scatter_128 = _make_edge_scatter(128, _B128, _NB128)
_edge_scatter_64 = _make_edge_scatter(64, _B64, _NB64)


@functools.partial(
    pl.kernel,
    out_type=jax.ShapeDtypeStruct((_NC, _N_ACC, 16), jnp.float32),
    mesh=_mesh,
    scratch_types=[
        pltpu.VMEM((_NB64, _B64), jnp.int32),
        pltpu.VMEM((_B64, 16), jnp.float32),
        pltpu.VMEM_SHARED((_N_ACC, 16), jnp.float32),
    ],
    compiler_params=pltpu.CompilerParams(use_tc_tiling_on_sc=False),
)
def _deg_count(dst_hbm, ones_hbm, zero_hbm, out_hbm, dst_v, ones_v, acc):
  cid = lax.axis_index("c")
  sid = lax.axis_index("s")
  wid = cid * _NS + sid
  pltpu.sync_copy(dst_hbm.at[wid], dst_v)
  pltpu.sync_copy(ones_hbm, ones_v)
  pltpu.sync_copy(zero_hbm, acc.at[pl.ds(sid * _ROWS_PT, _ROWS_PT)])
  plsc.subcore_barrier()

  def step(j, carry):
    pltpu.sync_copy(ones_v, acc.at[dst_v.at[j]], add=True)
    return carry

  lax.fori_loop(0, _NB64, step, 0)
  plsc.subcore_barrier()
  pltpu.sync_copy(
      acc.at[pl.ds(sid * _ROWS_PT, _ROWS_PT)],
      out_hbm.at[cid, pl.ds(sid * _ROWS_PT, _ROWS_PT)],
  )


_BN = 2000


def _stage_a_body(degp_ref, x_ref, w1_ref, g1_ref, dinv_ref):
  deg = degp_ref[0, :, 0:1] + degp_ref[1, :, 0:1] + 1.0
  dinv = lax.rsqrt(deg)
  dinv_ref[...] = jnp.broadcast_to(dinv, (_BN, 16))
  p1 = jnp.dot(x_ref[...], w1_ref[...], preferred_element_type=jnp.float32)
  g1_ref[...] = dinv * p1


def _stage_a(degp, x, w1):
  return pl.pallas_call(
      _stage_a_body,
      grid=(_N // _BN,),
      in_specs=[
          pl.BlockSpec((_NC, _BN, 16), lambda i: (0, i, 0)),
          pl.BlockSpec((_BN, 128), lambda i: (i, 0)),
          pl.BlockSpec((128, 128), lambda i: (0, 0)),
      ],
      out_specs=[
          pl.BlockSpec((_BN, 128), lambda i: (i, 0)),
          pl.BlockSpec((_BN, 16), lambda i: (i, 0)),
      ],
      out_shape=[
          jax.ShapeDtypeStruct((_N, 128), jnp.float32),
          jax.ShapeDtypeStruct((_N, 16), jnp.float32),
      ],
  )(degp, x, w1)


def _stage_b_body(s1_ref, g1_ref, dinv_ref, b1_ref, w2_ref, h_ref, g2_ref):
  dinv = dinv_ref[:, 0:1]
  h = jnp.maximum(dinv * (s1_ref[0] + s1_ref[1] + g1_ref[...]) + b1_ref[...], 0.0)
  h_ref[...] = h
  g2_ref[...] = dinv * jnp.dot(h, w2_ref[...], preferred_element_type=jnp.float32)


def _stage_b(s1, g1, dinv, b1, w2):
  return pl.pallas_call(
      _stage_b_body,
      grid=(_N // _BN,),
      in_specs=[
          pl.BlockSpec((_NC, _BN, 128), lambda i: (0, i, 0)),
          pl.BlockSpec((_BN, 128), lambda i: (i, 0)),
          pl.BlockSpec((_BN, 16), lambda i: (i, 0)),
          pl.BlockSpec((1, 128), lambda i: (0, 0)),
          pl.BlockSpec((128, 64), lambda i: (0, 0)),
      ],
      out_specs=[
          pl.BlockSpec((_BN, 128), lambda i: (i, 0)),
          pl.BlockSpec((_BN, 64), lambda i: (i, 0)),
      ],
      out_shape=[
          jax.ShapeDtypeStruct((_N, 128), jnp.float32),
          jax.ShapeDtypeStruct((_N, 64), jnp.float32),
      ],
  )(s1, g1, dinv, b1, w2)


def _stage_c_body(s2_ref, g2_ref, dinv_ref, b2_ref, x_ref, h_ref, wl_ref, bl_ref, out_ref):
  dinv = dinv_ref[:, 0:1]
  h2 = dinv * (s2_ref[0] + s2_ref[1] + g2_ref[...]) + b2_ref[...]
  wl = wl_ref[...]
  logits = (
      jnp.dot(x_ref[...], wl[0:128], preferred_element_type=jnp.float32)
      + jnp.dot(h_ref[...], wl[128:256], preferred_element_type=jnp.float32)
      + jnp.dot(h2, wl[256:320], preferred_element_type=jnp.float32)
      + bl_ref[...]
  )
  m = jnp.max(logits, axis=-1, keepdims=True)
  z = logits - m
  lse = jnp.log(jnp.sum(jnp.exp(z), axis=-1, keepdims=True))
  out_ref[...] = z - lse


def _stage_c(s2, g2, dinv, b2, x, h, wl, bl):
  return pl.pallas_call(
      _stage_c_body,
      grid=(_N // _BN,),
      in_specs=[
          pl.BlockSpec((_NC, _BN, 64), lambda i: (0, i, 0)),
          pl.BlockSpec((_BN, 64), lambda i: (i, 0)),
          pl.BlockSpec((_BN, 16), lambda i: (i, 0)),
          pl.BlockSpec((1, 64), lambda i: (0, 0)),
          pl.BlockSpec((_BN, 128), lambda i: (i, 0)),
          pl.BlockSpec((_BN, 128), lambda i: (i, 0)),
          pl.BlockSpec((320, 64), lambda i: (0, 0)),
          pl.BlockSpec((1, 64), lambda i: (0, 0)),
      ],
      out_specs=pl.BlockSpec((_BN, 64), lambda i: (i, 0)),
      out_shape=jax.ShapeDtypeStruct((_N, 64), jnp.float32),
  )(s2, g2, dinv, b2, x, h, wl, bl)


def _pad_edges(v, nb, batch, fill):
  ept = nb * batch
  pad = _NW * ept - _E
  return jnp.concatenate([v, jnp.full((pad,), fill, jnp.int32)]).reshape(_NW, nb, batch)


def kernel(x, edge_index, W1, b1, W2, b2, Wl, bl):
  src = edge_index[0]
  dst = edge_index[1]
  srcb128 = _pad_edges(src, _NB128, _B128, 0)
  dstb128 = _pad_edges(dst, _NB128, _B128, _N)
  srcb64 = _pad_edges(src, _NB64, _B64, 0)
  dstb64 = _pad_edges(dst, _NB64, _B64, _N)
  zeros128 = jnp.zeros((_ROWS_PT, 128), jnp.float32)
  zeros64 = jnp.zeros((_ROWS_PT, 64), jnp.float32)
  zeros16 = jnp.zeros((_ROWS_PT, 16), jnp.float32)
  ones16 = jnp.ones((_B64, 16), jnp.float32)

  degp = _deg_count(dstb64, ones16, zeros16)[:, :_N]
  g1, dinv = _stage_a(degp, x, W1)
  s1 = _edge_scatter_128(g1, srcb128, dstb128, zeros128)[:, :_N]
  h, g2 = _stage_b(s1, g1, dinv, b1.reshape(1, -1), W2)
  s2 = _edge_scatter_64(g2, srcb64, dstb64, zeros64)[:, :_N]
  return _stage_c(s2, g2, dinv, b2.reshape(1, -1), x, h, Wl, bl.reshape(1, -1))


# speedup vs baseline: 20.6687x; 1.4329x over previous
import functools

import jax
import jax.numpy as jnp
from jax import lax
from jax.experimental import pallas as pl
from jax.experimental.pallas import tpu as pltpu
from jax.experimental.pallas import tpu_sc as plsc

_N = 10000
_E = 320000
_NC = 2
_NS = 16
_NW = _NC * _NS
_ROWS_PT = 632
_N_ACC = _NS * _ROWS_PT

_B128, _NB128 = 112, 90
_B64, _NB64 = 128, 80
_mesh = plsc.VectorSubcoreMesh(
    core_axis_name="c", subcore_axis_name="s", num_cores=_NC, num_subcores=_NS
)


def _make_edge_scatter(d, batch, nb):
  @functools.partial(
      pl.kernel,
      out_type=jax.ShapeDtypeStruct((_NC, _N_ACC, d), jnp.float32),
      mesh=_mesh,
      scratch_types=[
          pltpu.VMEM((nb, batch), jnp.int32),
          pltpu.VMEM((nb, batch), jnp.int32),
          pltpu.VMEM((2, batch, d), jnp.float32),
          pltpu.VMEM_SHARED((_N_ACC, d), jnp.float32),
          pltpu.SemaphoreType.DMA,
          pltpu.SemaphoreType.DMA,
      ],
      compiler_params=pltpu.CompilerParams(use_tc_tiling_on_sc=False),
  )
  def k(g_hbm, src_hbm, dst_hbm, zero_hbm, out_hbm, src_v, dst_v, rows_v, acc,
        sem0, sem1):
    cid = lax.axis_index("c")
    sid = lax.axis_index("s")
    wid = cid * _NS + sid
    pltpu.sync_copy(src_hbm.at[wid], src_v)
    pltpu.sync_copy(dst_hbm.at[wid], dst_v)
    pltpu.sync_copy(zero_hbm, acc.at[pl.ds(sid * _ROWS_PT, _ROWS_PT)])
    plsc.subcore_barrier()

    pltpu.async_copy(g_hbm.at[src_v.at[0]], rows_v.at[0], sem0)

    def step(i, carry):
      j0 = 2 * i
      j1 = j0 + 1
      pltpu.make_async_copy(g_hbm.at[src_v.at[j0]], rows_v.at[0], sem0).wait()
      pltpu.async_copy(g_hbm.at[src_v.at[j1]], rows_v.at[1], sem1)
      pltpu.sync_copy(rows_v.at[0], acc.at[dst_v.at[j0]], add=True)
      pltpu.make_async_copy(g_hbm.at[src_v.at[j1]], rows_v.at[1], sem1).wait()

      @pl.when(i + 1 < nb // 2)
      def _():
        pltpu.async_copy(g_hbm.at[src_v.at[j0 + 2]], rows_v.at[0], sem0)

      pltpu.sync_copy(rows_v.at[1], acc.at[dst_v.at[j1]], add=True)
      return carry

    lax.fori_loop(0, nb // 2, step, 0)
    plsc.subcore_barrier()
    pltpu.sync_copy(
        acc.at[pl.ds(sid * _ROWS_PT, _ROWS_PT)],
        out_hbm.at[cid, pl.ds(sid * _ROWS_PT, _ROWS_PT)],
    )

  return k


_edge_scatter_128 = _make_edge_scatter(128, _B128, _NB128)
_edge_scatter_64 = _make_edge_scatter(64, _B64, _NB64)

def _make_edge_scatter_spmem(d, batch, nb):
  @functools.partial(
      pl.kernel,
      out_type=jax.ShapeDtypeStruct((_NC, _N_ACC, d), jnp.float32),
      mesh=_mesh,
      scratch_types=[
          pltpu.VMEM((nb, batch), jnp.int32),
          pltpu.VMEM((nb, batch), jnp.int32),
          pltpu.VMEM((2, batch, d), jnp.float32),
          pltpu.VMEM_SHARED((_N_ACC, d), jnp.float32),
          pltpu.VMEM_SHARED((_N_ACC, d), jnp.float32),
          pltpu.SemaphoreType.DMA,
          pltpu.SemaphoreType.DMA,
      ],
      compiler_params=pltpu.CompilerParams(use_tc_tiling_on_sc=False),
  )
  def k(g_hbm, src_hbm, dst_hbm, zero_hbm, out_hbm, src_v, dst_v, rows_v,
        g_sh, acc, sem0, sem1):
    cid = lax.axis_index("c")
    sid = lax.axis_index("s")
    wid = cid * _NS + sid
    pltpu.sync_copy(src_hbm.at[wid], src_v)
    pltpu.sync_copy(dst_hbm.at[wid], dst_v)
    pltpu.sync_copy(g_hbm.at[pl.ds(sid * _ROWS_PT, _ROWS_PT)],
                    g_sh.at[pl.ds(sid * _ROWS_PT, _ROWS_PT)])
    pltpu.sync_copy(zero_hbm, acc.at[pl.ds(sid * _ROWS_PT, _ROWS_PT)])
    plsc.subcore_barrier()

    pltpu.async_copy(g_sh.at[src_v.at[0]], rows_v.at[0], sem0)

    def step(i, carry):
      j0 = 2 * i
      j1 = j0 + 1
      pltpu.make_async_copy(g_sh.at[src_v.at[j0]], rows_v.at[0], sem0).wait()
      pltpu.async_copy(g_sh.at[src_v.at[j1]], rows_v.at[1], sem1)
      pltpu.sync_copy(rows_v.at[0], acc.at[dst_v.at[j0]], add=True)
      pltpu.make_async_copy(g_sh.at[src_v.at[j1]], rows_v.at[1], sem1).wait()

      @pl.when(i + 1 < nb // 2)
      def _():
        pltpu.async_copy(g_sh.at[src_v.at[j0 + 2]], rows_v.at[0], sem0)

      pltpu.sync_copy(rows_v.at[1], acc.at[dst_v.at[j1]], add=True)
      return carry

    lax.fori_loop(0, nb // 2, step, 0)
    plsc.subcore_barrier()
    pltpu.sync_copy(
        acc.at[pl.ds(sid * _ROWS_PT, _ROWS_PT)],
        out_hbm.at[cid, pl.ds(sid * _ROWS_PT, _ROWS_PT)],
    )

  return k


_edge_scatter_64s = _make_edge_scatter_spmem(64, _B64, _NB64)


@functools.partial(
    pl.kernel,
    out_type=jax.ShapeDtypeStruct((_NC, _N_ACC, 16), jnp.float32),
    mesh=_mesh,
    scratch_types=[
        pltpu.VMEM((_NB64, _B64), jnp.int32),
        pltpu.VMEM((_B64, 16), jnp.float32),
        pltpu.VMEM_SHARED((_N_ACC, 16), jnp.float32),
    ],
    compiler_params=pltpu.CompilerParams(use_tc_tiling_on_sc=False),
)
def _deg_count(dst_hbm, ones_hbm, zero_hbm, out_hbm, dst_v, ones_v, acc):
  cid = lax.axis_index("c")
  sid = lax.axis_index("s")
  wid = cid * _NS + sid
  pltpu.sync_copy(dst_hbm.at[wid], dst_v)
  pltpu.sync_copy(ones_hbm, ones_v)
  pltpu.sync_copy(zero_hbm, acc.at[pl.ds(sid * _ROWS_PT, _ROWS_PT)])
  plsc.subcore_barrier()

  def step(j, carry):
    pltpu.sync_copy(ones_v, acc.at[dst_v.at[j]], add=True)
    return carry

  lax.fori_loop(0, _NB64, step, 0)
  plsc.subcore_barrier()
  pltpu.sync_copy(
      acc.at[pl.ds(sid * _ROWS_PT, _ROWS_PT)],
      out_hbm.at[cid, pl.ds(sid * _ROWS_PT, _ROWS_PT)],
  )


_BN = 2000


def _stage_a_body(degp_ref, x_ref, w1_ref, g1_ref, dinv_ref):
  deg = degp_ref[0, :, 0:1] + degp_ref[1, :, 0:1] + 1.0
  dinv = lax.rsqrt(deg)
  dinv_ref[...] = jnp.broadcast_to(dinv, (_BN, 16))
  p1 = jnp.dot(x_ref[...], w1_ref[...], preferred_element_type=jnp.float32)
  g1_ref[...] = dinv * p1


def _stage_a(degp, x, w1):
  return pl.pallas_call(
      _stage_a_body,
      grid=(_N // _BN,),
      in_specs=[
          pl.BlockSpec((_NC, _BN, 16), lambda i: (0, i, 0)),
          pl.BlockSpec((_BN, 128), lambda i: (i, 0)),
          pl.BlockSpec((128, 128), lambda i: (0, 0)),
      ],
      out_specs=[
          pl.BlockSpec((_BN, 128), lambda i: (i, 0)),
          pl.BlockSpec((_BN, 16), lambda i: (i, 0)),
      ],
      out_shape=[
          jax.ShapeDtypeStruct((_N, 128), jnp.float32),
          jax.ShapeDtypeStruct((_N, 16), jnp.float32),
      ],
  )(degp, x, w1)


def _stage_b_body(s1_ref, g1_ref, dinv_ref, b1_ref, w2_ref, h_ref, g2_ref):
  dinv = dinv_ref[:, 0:1]
  h = jnp.maximum(dinv * (s1_ref[0] + s1_ref[1] + g1_ref[...]) + b1_ref[...], 0.0)
  h_ref[...] = h
  g2_ref[...] = dinv * jnp.dot(h, w2_ref[...], preferred_element_type=jnp.float32)


def _stage_b(s1, g1, dinv, b1, w2):
  return pl.pallas_call(
      _stage_b_body,
      grid=(_N // _BN,),
      in_specs=[
          pl.BlockSpec((_NC, _BN, 128), lambda i: (0, i, 0)),
          pl.BlockSpec((_BN, 128), lambda i: (i, 0)),
          pl.BlockSpec((_BN, 16), lambda i: (i, 0)),
          pl.BlockSpec((1, 128), lambda i: (0, 0)),
          pl.BlockSpec((128, 64), lambda i: (0, 0)),
      ],
      out_specs=[
          pl.BlockSpec((_BN, 128), lambda i: (i, 0)),
          pl.BlockSpec((_BN, 64), lambda i: (i, 0)),
      ],
      out_shape=[
          jax.ShapeDtypeStruct((_N, 128), jnp.float32),
          jax.ShapeDtypeStruct((_N, 64), jnp.float32),
      ],
  )(s1, g1, dinv, b1, w2)


def _stage_c_body(s2_ref, g2_ref, dinv_ref, b2_ref, x_ref, h_ref, wl_ref, bl_ref, out_ref):
  dinv = dinv_ref[:, 0:1]
  h2 = dinv * (s2_ref[0] + s2_ref[1] + g2_ref[...]) + b2_ref[...]
  wl = wl_ref[...]
  logits = (
      jnp.dot(x_ref[...], wl[0:128], preferred_element_type=jnp.float32)
      + jnp.dot(h_ref[...], wl[128:256], preferred_element_type=jnp.float32)
      + jnp.dot(h2, wl[256:320], preferred_element_type=jnp.float32)
      + bl_ref[...]
  )
  m = jnp.max(logits, axis=-1, keepdims=True)
  z = logits - m
  lse = jnp.log(jnp.sum(jnp.exp(z), axis=-1, keepdims=True))
  out_ref[...] = z - lse


def _stage_c(s2, g2, dinv, b2, x, h, wl, bl):
  return pl.pallas_call(
      _stage_c_body,
      grid=(_N // _BN,),
      in_specs=[
          pl.BlockSpec((_NC, _BN, 64), lambda i: (0, i, 0)),
          pl.BlockSpec((_BN, 64), lambda i: (i, 0)),
          pl.BlockSpec((_BN, 16), lambda i: (i, 0)),
          pl.BlockSpec((1, 64), lambda i: (0, 0)),
          pl.BlockSpec((_BN, 128), lambda i: (i, 0)),
          pl.BlockSpec((_BN, 128), lambda i: (i, 0)),
          pl.BlockSpec((320, 64), lambda i: (0, 0)),
          pl.BlockSpec((1, 64), lambda i: (0, 0)),
      ],
      out_specs=pl.BlockSpec((_BN, 64), lambda i: (i, 0)),
      out_shape=jax.ShapeDtypeStruct((_N, 64), jnp.float32),
  )(s2, g2, dinv, b2, x, h, wl, bl)


def _pad_edges(v, nb, batch, fill):
  ept = nb * batch
  pad = _NW * ept - _E
  return jnp.concatenate([v, jnp.full((pad,), fill, jnp.int32)]).reshape(_NW, nb, batch)


def kernel(x, edge_index, W1, b1, W2, b2, Wl, bl):
  src = edge_index[0]
  dst = edge_index[1]
  srcb128 = _pad_edges(src, _NB128, _B128, 0)
  dstb128 = _pad_edges(dst, _NB128, _B128, _N)
  srcb64 = _pad_edges(src, _NB64, _B64, 0)
  dstb64 = _pad_edges(dst, _NB64, _B64, _N)
  zeros128 = jnp.zeros((_ROWS_PT, 128), jnp.float32)
  zeros64 = jnp.zeros((_ROWS_PT, 64), jnp.float32)
  zeros16 = jnp.zeros((_ROWS_PT, 16), jnp.float32)
  ones16 = jnp.ones((_B64, 16), jnp.float32)

  degp = _deg_count(dstb64, ones16, zeros16)[:, :_N]
  g1, dinv = _stage_a(degp, x, W1)
  s1 = _edge_scatter_128(g1, srcb128, dstb128, zeros128)[:, :_N]
  h, g2 = _stage_b(s1, g1, dinv, b1.reshape(1, -1), W2)
  g2p = jnp.pad(g2, ((0, _N_ACC - _N), (0, 0)))
  s2 = _edge_scatter_64s(g2p, srcb64, dstb64, zeros64)[:, :_N]
  return _stage_c(s2, g2, dinv, b2.reshape(1, -1), x, h, Wl, bl.reshape(1, -1))


# speedup vs baseline: 21.5937x; 1.0448x over previous
import functools

import jax
import jax.numpy as jnp
from jax import lax
from jax.experimental import pallas as pl
from jax.experimental.pallas import tpu as pltpu
from jax.experimental.pallas import tpu_sc as plsc

_N = 10000
_E = 320000
_NC = 2
_NS = 16
_NW = _NC * _NS
_ROWS_PT = 632
_N_ACC = _NS * _ROWS_PT

_B128, _NB128 = 112, 90
_B64, _NB64 = 128, 80
_mesh = plsc.VectorSubcoreMesh(
    core_axis_name="c", subcore_axis_name="s", num_cores=_NC, num_subcores=_NS
)


def _make_edge_scatter(d, batch, nb):
  @functools.partial(
      pl.kernel,
      out_type=jax.ShapeDtypeStruct((_NC, _N_ACC, d), jnp.float32),
      mesh=_mesh,
      scratch_types=[
          pltpu.VMEM((nb, batch), jnp.int32),
          pltpu.VMEM((nb, batch), jnp.int32),
          pltpu.VMEM((2, batch, d), jnp.float32),
          pltpu.VMEM_SHARED((_N_ACC, d), jnp.float32),
          pltpu.SemaphoreType.DMA,
          pltpu.SemaphoreType.DMA,
      ],
      compiler_params=pltpu.CompilerParams(use_tc_tiling_on_sc=False),
  )
  def k(g_hbm, src_hbm, dst_hbm, zero_hbm, out_hbm, src_v, dst_v, rows_v, acc,
        sem0, sem1):
    cid = lax.axis_index("c")
    sid = lax.axis_index("s")
    wid = cid * _NS + sid
    pltpu.sync_copy(src_hbm.at[wid], src_v)
    pltpu.sync_copy(dst_hbm.at[wid], dst_v)
    pltpu.sync_copy(zero_hbm, acc.at[pl.ds(sid * _ROWS_PT, _ROWS_PT)])
    plsc.subcore_barrier()

    pltpu.async_copy(g_hbm.at[src_v.at[0]], rows_v.at[0], sem0)

    def step(i, carry):
      j0 = 2 * i
      j1 = j0 + 1
      pltpu.make_async_copy(g_hbm.at[src_v.at[j0]], rows_v.at[0], sem0).wait()
      pltpu.async_copy(g_hbm.at[src_v.at[j1]], rows_v.at[1], sem1)
      pltpu.sync_copy(rows_v.at[0], acc.at[dst_v.at[j0]], add=True)
      pltpu.make_async_copy(g_hbm.at[src_v.at[j1]], rows_v.at[1], sem1).wait()

      @pl.when(i + 1 < nb // 2)
      def _():
        pltpu.async_copy(g_hbm.at[src_v.at[j0 + 2]], rows_v.at[0], sem0)

      pltpu.sync_copy(rows_v.at[1], acc.at[dst_v.at[j1]], add=True)
      return carry

    lax.fori_loop(0, nb // 2, step, 0)
    plsc.subcore_barrier()
    pltpu.sync_copy(
        acc.at[pl.ds(sid * _ROWS_PT, _ROWS_PT)],
        out_hbm.at[cid, pl.ds(sid * _ROWS_PT, _ROWS_PT)],
    )

  return k


_edge_scatter_128 = _make_edge_scatter(128, _B128, _NB128)
_edge_scatter_64 = _make_edge_scatter(64, _B64, _NB64)

def _make_edge_scatter_spmem(d, batch, nb):
  @functools.partial(
      pl.kernel,
      out_type=jax.ShapeDtypeStruct((_NC, _N_ACC, d), jnp.float32),
      mesh=_mesh,
      scratch_types=[
          pltpu.VMEM((nb, batch), jnp.int32),
          pltpu.VMEM((nb, batch), jnp.int32),
          pltpu.VMEM((2, batch, d), jnp.float32),
          pltpu.VMEM_SHARED((_N_ACC, d), jnp.float32),
          pltpu.VMEM_SHARED((_N_ACC, d), jnp.float32),
          pltpu.SemaphoreType.DMA,
          pltpu.SemaphoreType.DMA,
      ],
      compiler_params=pltpu.CompilerParams(use_tc_tiling_on_sc=False),
  )
  def k(g_hbm, src_hbm, dst_hbm, zero_hbm, out_hbm, src_v, dst_v, rows_v,
        g_sh, acc, sem0, sem1):
    cid = lax.axis_index("c")
    sid = lax.axis_index("s")
    wid = cid * _NS + sid
    pltpu.sync_copy(src_hbm.at[wid], src_v)
    pltpu.sync_copy(dst_hbm.at[wid], dst_v)
    pltpu.sync_copy(g_hbm.at[pl.ds(sid * _ROWS_PT, _ROWS_PT)],
                    g_sh.at[pl.ds(sid * _ROWS_PT, _ROWS_PT)])
    pltpu.sync_copy(zero_hbm, acc.at[pl.ds(sid * _ROWS_PT, _ROWS_PT)])
    plsc.subcore_barrier()

    pltpu.async_copy(g_sh.at[src_v.at[0]], rows_v.at[0], sem0)

    def step(i, carry):
      j0 = 2 * i
      j1 = j0 + 1
      pltpu.make_async_copy(g_sh.at[src_v.at[j0]], rows_v.at[0], sem0).wait()
      pltpu.async_copy(g_sh.at[src_v.at[j1]], rows_v.at[1], sem1)
      pltpu.sync_copy(rows_v.at[0], acc.at[dst_v.at[j0]], add=True)
      pltpu.make_async_copy(g_sh.at[src_v.at[j1]], rows_v.at[1], sem1).wait()

      @pl.when(i + 1 < nb // 2)
      def _():
        pltpu.async_copy(g_sh.at[src_v.at[j0 + 2]], rows_v.at[0], sem0)

      pltpu.sync_copy(rows_v.at[1], acc.at[dst_v.at[j1]], add=True)
      return carry

    lax.fori_loop(0, nb // 2, step, 0)
    plsc.subcore_barrier()
    pltpu.sync_copy(
        acc.at[pl.ds(sid * _ROWS_PT, _ROWS_PT)],
        out_hbm.at[cid, pl.ds(sid * _ROWS_PT, _ROWS_PT)],
    )

  return k


_edge_scatter_64s = _make_edge_scatter_spmem(64, _B64, _NB64)

def _make_edge_scatter_spmem2(nb, batch):
  @functools.partial(
      pl.kernel,
      out_type=jax.ShapeDtypeStruct((_NC, 2, _N_ACC, 64), jnp.float32),
      mesh=_mesh,
      scratch_types=[
          pltpu.VMEM((nb, batch), jnp.int32),
          pltpu.VMEM((nb, batch), jnp.int32),
          pltpu.VMEM((2, batch, 64), jnp.float32),
          pltpu.VMEM_SHARED((_N_ACC, 64), jnp.float32),
          pltpu.VMEM_SHARED((_N_ACC, 64), jnp.float32),
          pltpu.SemaphoreType.DMA,
          pltpu.SemaphoreType.DMA,
      ],
      compiler_params=pltpu.CompilerParams(use_tc_tiling_on_sc=False),
  )
  def k(g_hbm, src_hbm, dst_hbm, zero_hbm, out_hbm, src_v, dst_v, rows_v,
        g_sh, acc, sem0, sem1):
    cid = lax.axis_index("c")
    sid = lax.axis_index("s")
    wid = cid * _NS + sid
    pltpu.sync_copy(src_hbm.at[wid], src_v)
    pltpu.sync_copy(dst_hbm.at[wid], dst_v)
    for p in range(2):
      pltpu.sync_copy(g_hbm.at[p, pl.ds(sid * _ROWS_PT, _ROWS_PT)],
                      g_sh.at[pl.ds(sid * _ROWS_PT, _ROWS_PT)])
      pltpu.sync_copy(zero_hbm, acc.at[pl.ds(sid * _ROWS_PT, _ROWS_PT)])
      plsc.subcore_barrier()

      pltpu.async_copy(g_sh.at[src_v.at[0]], rows_v.at[0], sem0)

      def step(i, carry):
        j0 = 2 * i
        j1 = j0 + 1
        pltpu.make_async_copy(g_sh.at[src_v.at[j0]], rows_v.at[0], sem0).wait()
        pltpu.async_copy(g_sh.at[src_v.at[j1]], rows_v.at[1], sem1)
        pltpu.sync_copy(rows_v.at[0], acc.at[dst_v.at[j0]], add=True)
        pltpu.make_async_copy(g_sh.at[src_v.at[j1]], rows_v.at[1], sem1).wait()

        @pl.when(i + 1 < nb // 2)
        def _():
          pltpu.async_copy(g_sh.at[src_v.at[j0 + 2]], rows_v.at[0], sem0)

        pltpu.sync_copy(rows_v.at[1], acc.at[dst_v.at[j1]], add=True)
        return carry

      lax.fori_loop(0, nb // 2, step, 0)
      plsc.subcore_barrier()
      pltpu.sync_copy(
          acc.at[pl.ds(sid * _ROWS_PT, _ROWS_PT)],
          out_hbm.at[cid, p, pl.ds(sid * _ROWS_PT, _ROWS_PT)],
      )

  return k


_edge_scatter_128s = _make_edge_scatter_spmem2(_NB64, _B64)


@functools.partial(
    pl.kernel,
    out_type=jax.ShapeDtypeStruct((_NC, _N_ACC, 16), jnp.float32),
    mesh=_mesh,
    scratch_types=[
        pltpu.VMEM((_NB64, _B64), jnp.int32),
        pltpu.VMEM((_B64, 16), jnp.float32),
        pltpu.VMEM_SHARED((_N_ACC, 16), jnp.float32),
    ],
    compiler_params=pltpu.CompilerParams(use_tc_tiling_on_sc=False),
)
def _deg_count(dst_hbm, ones_hbm, zero_hbm, out_hbm, dst_v, ones_v, acc):
  cid = lax.axis_index("c")
  sid = lax.axis_index("s")
  wid = cid * _NS + sid
  pltpu.sync_copy(dst_hbm.at[wid], dst_v)
  pltpu.sync_copy(ones_hbm, ones_v)
  pltpu.sync_copy(zero_hbm, acc.at[pl.ds(sid * _ROWS_PT, _ROWS_PT)])
  plsc.subcore_barrier()

  def step(j, carry):
    pltpu.sync_copy(ones_v, acc.at[dst_v.at[j]], add=True)
    return carry

  lax.fori_loop(0, _NB64, step, 0)
  plsc.subcore_barrier()
  pltpu.sync_copy(
      acc.at[pl.ds(sid * _ROWS_PT, _ROWS_PT)],
      out_hbm.at[cid, pl.ds(sid * _ROWS_PT, _ROWS_PT)],
  )


_BN = 2000


def _stage_a_body(degp_ref, x_ref, w1_ref, g1_ref, dinv_ref):
  deg = degp_ref[0, :, 0:1] + degp_ref[1, :, 0:1] + 1.0
  dinv = lax.rsqrt(deg)
  dinv_ref[...] = jnp.broadcast_to(dinv, (_BN, 16))
  p1 = jnp.dot(x_ref[...], w1_ref[...], preferred_element_type=jnp.float32)
  g1_ref[...] = dinv * p1


def _stage_a(degp, x, w1):
  return pl.pallas_call(
      _stage_a_body,
      grid=(_N // _BN,),
      in_specs=[
          pl.BlockSpec((_NC, _BN, 16), lambda i: (0, i, 0)),
          pl.BlockSpec((_BN, 128), lambda i: (i, 0)),
          pl.BlockSpec((128, 128), lambda i: (0, 0)),
      ],
      out_specs=[
          pl.BlockSpec((_BN, 128), lambda i: (i, 0)),
          pl.BlockSpec((_BN, 16), lambda i: (i, 0)),
      ],
      out_shape=[
          jax.ShapeDtypeStruct((_N, 128), jnp.float32),
          jax.ShapeDtypeStruct((_N, 16), jnp.float32),
      ],
  )(degp, x, w1)


def _stage_b_body(s1_ref, g1_ref, dinv_ref, b1_ref, w2_ref, h_ref, g2_ref):
  dinv = dinv_ref[:, 0:1]
  s1 = jnp.concatenate(
      [s1_ref[0, 0] + s1_ref[1, 0], s1_ref[0, 1] + s1_ref[1, 1]], axis=-1)
  h = jnp.maximum(dinv * (s1 + g1_ref[...]) + b1_ref[...], 0.0)
  h_ref[...] = h
  g2_ref[...] = dinv * jnp.dot(h, w2_ref[...], preferred_element_type=jnp.float32)


def _stage_b(s1, g1, dinv, b1, w2):
  return pl.pallas_call(
      _stage_b_body,
      grid=(_N // _BN,),
      in_specs=[
          pl.BlockSpec((_NC, 2, _BN, 64), lambda i: (0, 0, i, 0)),
          pl.BlockSpec((_BN, 128), lambda i: (i, 0)),
          pl.BlockSpec((_BN, 16), lambda i: (i, 0)),
          pl.BlockSpec((1, 128), lambda i: (0, 0)),
          pl.BlockSpec((128, 64), lambda i: (0, 0)),
      ],
      out_specs=[
          pl.BlockSpec((_BN, 128), lambda i: (i, 0)),
          pl.BlockSpec((_BN, 64), lambda i: (i, 0)),
      ],
      out_shape=[
          jax.ShapeDtypeStruct((_N, 128), jnp.float32),
          jax.ShapeDtypeStruct((_N, 64), jnp.float32),
      ],
  )(s1, g1, dinv, b1, w2)


def _stage_c_body(s2_ref, g2_ref, dinv_ref, b2_ref, x_ref, h_ref, wl_ref, bl_ref, out_ref):
  dinv = dinv_ref[:, 0:1]
  h2 = dinv * (s2_ref[0] + s2_ref[1] + g2_ref[...]) + b2_ref[...]
  wl = wl_ref[...]
  logits = (
      jnp.dot(x_ref[...], wl[0:128], preferred_element_type=jnp.float32)
      + jnp.dot(h_ref[...], wl[128:256], preferred_element_type=jnp.float32)
      + jnp.dot(h2, wl[256:320], preferred_element_type=jnp.float32)
      + bl_ref[...]
  )
  m = jnp.max(logits, axis=-1, keepdims=True)
  z = logits - m
  lse = jnp.log(jnp.sum(jnp.exp(z), axis=-1, keepdims=True))
  out_ref[...] = z - lse


def _stage_c(s2, g2, dinv, b2, x, h, wl, bl):
  return pl.pallas_call(
      _stage_c_body,
      grid=(_N // _BN,),
      in_specs=[
          pl.BlockSpec((_NC, _BN, 64), lambda i: (0, i, 0)),
          pl.BlockSpec((_BN, 64), lambda i: (i, 0)),
          pl.BlockSpec((_BN, 16), lambda i: (i, 0)),
          pl.BlockSpec((1, 64), lambda i: (0, 0)),
          pl.BlockSpec((_BN, 128), lambda i: (i, 0)),
          pl.BlockSpec((_BN, 128), lambda i: (i, 0)),
          pl.BlockSpec((320, 64), lambda i: (0, 0)),
          pl.BlockSpec((1, 64), lambda i: (0, 0)),
      ],
      out_specs=pl.BlockSpec((_BN, 64), lambda i: (i, 0)),
      out_shape=jax.ShapeDtypeStruct((_N, 64), jnp.float32),
  )(s2, g2, dinv, b2, x, h, wl, bl)


def _pad_edges(v, nb, batch, fill):
  ept = nb * batch
  pad = _NW * ept - _E
  return jnp.concatenate([v, jnp.full((pad,), fill, jnp.int32)]).reshape(_NW, nb, batch)


def kernel(x, edge_index, W1, b1, W2, b2, Wl, bl):
  src = edge_index[0]
  dst = edge_index[1]
  srcb64 = _pad_edges(src, _NB64, _B64, 0)
  dstb64 = _pad_edges(dst, _NB64, _B64, _N)
  zeros64 = jnp.zeros((_ROWS_PT, 64), jnp.float32)
  zeros16 = jnp.zeros((_ROWS_PT, 16), jnp.float32)
  ones16 = jnp.ones((_B64, 16), jnp.float32)

  degp = _deg_count(dstb64, ones16, zeros16)[:, :_N]
  g1, dinv = _stage_a(degp, x, W1)
  g1p = jnp.pad(g1, ((0, _N_ACC - _N), (0, 0)))
  g1s = jnp.stack([g1p[:, :64], g1p[:, 64:]])
  s1 = _edge_scatter_128s(g1s, srcb64, dstb64, zeros64)[:, :, :_N]
  h, g2 = _stage_b(s1, g1, dinv, b1.reshape(1, -1), W2)
  g2p = jnp.pad(g2, ((0, _N_ACC - _N), (0, 0)))
  s2 = _edge_scatter_64s(g2p, srcb64, dstb64, zeros64)[:, :_N]
  return _stage_c(s2, g2, dinv, b2.reshape(1, -1), x, h, Wl, bl.reshape(1, -1))
